# baseline (speedup 1.0000x reference)
import functools

import jax
import jax.numpy as jnp
from jax import lax
from jax.experimental import pallas as pl
from jax.experimental.pallas import tpu as pltpu
from jax.experimental.pallas import tpu_sc as plsc

_VOCAB = 1000000
_D = 64
_LANES = 16
_NV = _D // _LANES

_NC = 2
_NS = 16
_NW = _NC * _NS

_CHUNK = 128


def _make_gather_sum(n_tokens: int):
    per_w = n_tokens // _NW
    nch = per_w // _CHUNK

    mesh = plsc.VectorSubcoreMesh(core_axis_name="c", subcore_axis_name="s")

    @functools.partial(
        pl.kernel,
        out_type=jax.ShapeDtypeStruct((_NW, _D), jnp.float32),
        mesh=mesh,
        scratch_types=[
            pltpu.VMEM((nch, _CHUNK), jnp.int32),
            pltpu.VMEM((nch, _CHUNK, _D), jnp.float32),
            pltpu.VMEM((_D,), jnp.float32),
            pltpu.SemaphoreType.DMA,
            pltpu.SemaphoreType.DMA((nch,)),
            pltpu.SemaphoreType.DMA,
        ],
        compiler_params=pltpu.CompilerParams(use_tc_tiling_on_sc=False),
    )
    def gather_sum(idx_hbm, table_hbm, out_hbm, idx_v, rows_v, acc_v,
                   sem_i, sem_g, sem_o):
        wid = lax.axis_index("s") * _NC + lax.axis_index("c")

        pltpu.async_copy(idx_hbm.at[wid], idx_v, sem_i).wait()

        copies = []
        for j in range(nch):
            copies.append(
                pltpu.async_copy(
                    table_hbm.at[idx_v.at[j]], rows_v.at[j], sem_g.at[j]
                )
            )

        acc = [jnp.zeros((_LANES,), jnp.float32) for _ in range(_NV)]
        for j in range(nch):
            copies[j].wait()

            def body(i, carry):
                new = []
                for k in range(_NV):
                    r = rows_v[j, i, pl.ds(k * _LANES, _LANES)]
                    new.append(carry[k] + r)
                return tuple(new)

            acc = lax.fori_loop(0, _CHUNK, body, tuple(acc))

        for k in range(_NV):
            acc_v[pl.ds(k * _LANES, _LANES)] = acc[k]
        pltpu.async_copy(acc_v, out_hbm.at[wid], sem_o).wait()

    return gather_sum


def _head_kernel(p_ref, wt_ref, b_ref, o_ref):
    bow = jnp.sum(p_ref[...], axis=0, keepdims=True)
    logits = (
        jnp.dot(bow, wt_ref[...], preferred_element_type=jnp.float32)
        + b_ref[...]
    )
    m = jnp.max(logits, axis=1, keepdims=True)
    e = jnp.exp(logits - m)
    lse = jnp.log(jnp.sum(e, axis=1, keepdims=True)) + m
    o_ref[...] = logits - lse


def kernel(inputs, emb_table, W, b):
    n_tokens = inputs.shape[0]
    num_labels = W.shape[0]

    per_w = n_tokens // _NW
    nch = per_w // _CHUNK
    idx = inputs.astype(jnp.int32).reshape(_NW, nch, _CHUNK)

    partials = _make_gather_sum(n_tokens)(idx, emb_table)

    out = pl.pallas_call(
        _head_kernel,
        out_shape=jax.ShapeDtypeStruct((1, num_labels), jnp.float32),
    )(partials, W.T, b.reshape(1, num_labels))
    return out


# speedup vs baseline: 1.7071x; 1.7071x over previous
import functools

import jax
import jax.numpy as jnp
from jax import lax
from jax.experimental import pallas as pl
from jax.experimental.pallas import tpu as pltpu
from jax.experimental.pallas import tpu_sc as plsc

_D = 64
_LANES = 16
_NV = _D // _LANES

_NC = 2
_NS = 16
_NW = _NC * _NS

_BANK = 64


def _make_gather_sum(n_tokens: int):
    per_w = n_tokens // _NW
    nbanks = per_w // _BANK

    mesh = plsc.VectorSubcoreMesh(core_axis_name="c", subcore_axis_name="s")

    @functools.partial(
        pl.kernel,
        out_type=jax.ShapeDtypeStruct((_NW, _D), jnp.float32),
        mesh=mesh,
        scratch_types=[
            pltpu.VMEM((per_w,), jnp.int32),
            pltpu.VMEM((2, _BANK, _D), jnp.float32),
            pltpu.VMEM((_D,), jnp.float32),
            pltpu.SemaphoreType.DMA,
            pltpu.SemaphoreType.DMA((2,)),
            pltpu.SemaphoreType.DMA,
        ],
    )
    def gather_sum(idx_hbm, table_hbm, out_hbm, idx_v, banks_v, acc_v,
                   sem_i, sem_g, sem_o):
        wid = lax.axis_index("s") * _NC + lax.axis_index("c")

        pltpu.async_copy(idx_hbm.at[wid], idx_v, sem_i).wait()

        def fire(p):
            bank = banks_v.at[p % 2]
            sem = sem_g.at[p % 2]

            def body(g, _):
                vec = idx_v[pl.ds(p * _BANK + g * _LANES, _LANES)]
                for lane in range(_LANES):
                    row = vec[lane]
                    pltpu.async_copy(
                        table_hbm.at[row], bank.at[g * _LANES + lane], sem
                    )
                return 0

            lax.fori_loop(0, _BANK // _LANES, body, 0)

        def drain(p):
            pltpu.make_async_copy(
                table_hbm.at[pl.ds(0, _BANK)], banks_v.at[p % 2], sem_g.at[p % 2]
            ).wait()

        fire(0)
        acc = [jnp.zeros((_LANES,), jnp.float32) for _ in range(_NV)]
        for p in range(nbanks):
            if p + 1 < nbanks:
                fire(p + 1)
            drain(p)
            bank = p % 2

            def body(i, carry):
                new = []
                for k in range(_NV):
                    row = banks_v[bank, i, pl.ds(k * _LANES, _LANES)]
                    new.append(carry[k] + row)
                return tuple(new)

            acc = lax.fori_loop(0, _BANK, body, tuple(acc))

        for k in range(_NV):
            acc_v[pl.ds(k * _LANES, _LANES)] = acc[k]
        pltpu.async_copy(acc_v, out_hbm.at[wid], sem_o).wait()

    return gather_sum


def _head_kernel(p_ref, wt_ref, b_ref, o_ref):
    bow = jnp.sum(p_ref[...], axis=0, keepdims=True)
    logits = (
        jnp.dot(bow, wt_ref[...], preferred_element_type=jnp.float32)
        + b_ref[...]
    )
    m = jnp.max(logits, axis=1, keepdims=True)
    e = jnp.exp(logits - m)
    lse = jnp.log(jnp.sum(e, axis=1, keepdims=True)) + m
    o_ref[...] = logits - lse


def kernel(inputs, emb_table, W, b):
    n_tokens = inputs.shape[0]
    num_labels = W.shape[0]

    idx = inputs.astype(jnp.int32).reshape(_NW, n_tokens // _NW)

    partials = _make_gather_sum(n_tokens)(idx, emb_table)

    out = pl.pallas_call(
        _head_kernel,
        out_shape=jax.ShapeDtypeStruct((1, num_labels), jnp.float32),
    )(partials, W.T, b.reshape(1, num_labels))
    return out


# speedup vs baseline: 1.7079x; 1.0005x over previous
import functools

import jax
import jax.numpy as jnp
from jax import lax
from jax.experimental import pallas as pl
from jax.experimental.pallas import tpu as pltpu
from jax.experimental.pallas import tpu_sc as plsc

_D = 64
_LANES = 16
_NV = _D // _LANES

_NC = 2
_NS = 16
_NW = _NC * _NS

_BANK = 64


def _make_gather_sum(n_tokens: int):
    per_w = n_tokens // _NW
    nbanks = per_w // _BANK

    mesh = plsc.VectorSubcoreMesh(core_axis_name="c", subcore_axis_name="s")

    @functools.partial(
        pl.kernel,
        out_type=jax.ShapeDtypeStruct((_NW, _D), jnp.float32),
        mesh=mesh,
        scratch_types=[
            pltpu.VMEM((per_w,), jnp.int32),
            pltpu.VMEM((2, _BANK, _D), jnp.float32),
            pltpu.VMEM((_D,), jnp.float32),
            pltpu.SemaphoreType.DMA,
            pltpu.SemaphoreType.DMA((2,)),
            pltpu.SemaphoreType.DMA,
        ],
        compiler_params=pltpu.CompilerParams(use_tc_tiling_on_sc=True),
    )
    def gather_sum(idx_hbm, table_hbm, out_hbm, idx_v, banks_v, acc_v,
                   sem_i, sem_g, sem_o):
        wid = lax.axis_index("s") * _NC + lax.axis_index("c")

        pltpu.async_copy(idx_hbm.at[wid], idx_v, sem_i).wait()

        def fire(p):
            bank = banks_v.at[p % 2]
            sem = sem_g.at[p % 2]

            def body(g, _):
                vec = idx_v[pl.ds(p * _BANK + g * _LANES, _LANES)]
                for lane in range(_LANES):
                    row = vec[lane]
                    pltpu.async_copy(
                        table_hbm.at[row], bank.at[g * _LANES + lane], sem
                    )
                return 0

            lax.fori_loop(0, _BANK // _LANES, body, 0)

        def drain(p):
            pltpu.make_async_copy(
                table_hbm.at[pl.ds(0, _BANK)], banks_v.at[p % 2], sem_g.at[p % 2]
            ).wait()

        fire(0)
        acc = [jnp.zeros((_LANES,), jnp.float32) for _ in range(_NV)]
        for p in range(nbanks):
            if p + 1 < nbanks:
                fire(p + 1)
            drain(p)
            bank = p % 2

            def body(i, carry):
                new = []
                for k in range(_NV):
                    row = banks_v[bank, i, pl.ds(k * _LANES, _LANES)]
                    new.append(carry[k] + row)
                return tuple(new)

            acc = lax.fori_loop(0, _BANK, body, tuple(acc))

        for k in range(_NV):
            acc_v[pl.ds(k * _LANES, _LANES)] = acc[k]
        pltpu.async_copy(acc_v, out_hbm.at[wid], sem_o).wait()

    return gather_sum


def _head_kernel(p_ref, wt_ref, b_ref, o_ref):
    bow = jnp.sum(p_ref[...], axis=0, keepdims=True)
    logits = (
        jnp.dot(bow, wt_ref[...], preferred_element_type=jnp.float32)
        + b_ref[...]
    )
    m = jnp.max(logits, axis=1, keepdims=True)
    e = jnp.exp(logits - m)
    lse = jnp.log(jnp.sum(e, axis=1, keepdims=True)) + m
    o_ref[...] = logits - lse


def kernel(inputs, emb_table, W, b):
    n_tokens = inputs.shape[0]
    num_labels = W.shape[0]

    idx = inputs.astype(jnp.int32).reshape(_NW, n_tokens // _NW)

    partials = _make_gather_sum(n_tokens)(idx, emb_table)

    out = pl.pallas_call(
        _head_kernel,
        out_shape=jax.ShapeDtypeStruct((1, num_labels), jnp.float32),
    )(partials, W.T, b.reshape(1, num_labels))
    return out


# speedup vs baseline: 4.7862x; 2.8023x over previous
import functools

import jax
import jax.numpy as jnp
from jax import lax
from jax.experimental import pallas as pl
from jax.experimental.pallas import tpu as pltpu
from jax.experimental.pallas import tpu_sc as plsc

_D = 64
_LANES = 16

_NC = 2
_NS = 16
_NW = _NC * _NS

_SCHUNK = 128

_CK = 16384


def _make_count(n_tokens: int, vocab: int):
    per_w = n_tokens // _NW
    nch = per_w // _SCHUNK

    mesh = plsc.VectorSubcoreMesh(core_axis_name="c", subcore_axis_name="s")

    @functools.partial(
        pl.kernel,
        out_type=jax.ShapeDtypeStruct((_NC, vocab), jnp.float32),
        mesh=mesh,
        scratch_types=[
            pltpu.VMEM((nch, _SCHUNK), jnp.int32),
            pltpu.VMEM((_SCHUNK,), jnp.float32),
            pltpu.VMEM_SHARED((vocab,), jnp.float32),
            pltpu.SemaphoreType.DMA,
        ],
    )
    def count(idx_hbm, zeros_hbm, out_hbm, idx_v, ones_v, cnt_sh, sem_i):
        cid = lax.axis_index("c")
        sid = lax.axis_index("s")
        wid = sid * _NC + cid

        ci = pltpu.async_copy(idx_hbm.at[wid], idx_v, sem_i)
        for g in range(_SCHUNK // _LANES):
            ones_v[pl.ds(g * _LANES, _LANES)] = jnp.ones(
                (_LANES,), jnp.float32
            )

        @pl.when(sid == 0)
        def _():
            pltpu.sync_copy(zeros_hbm, cnt_sh)

        ci.wait()
        plsc.subcore_barrier()

        for j in range(nch):
            pltpu.sync_copy(ones_v, cnt_sh.at[idx_v.at[j]], add=True)

        plsc.subcore_barrier()

        @pl.when(sid == 0)
        def _():
            pltpu.sync_copy(cnt_sh, out_hbm.at[cid])

    return count


def _matvec_head_kernel(nblk, vocab, cnt_ref, t_ref, w_ref, b_ref, o_ref,
                        acc_ref):
    k = pl.program_id(0)
    csum = cnt_ref[0:1, :] + cnt_ref[1:2, :]
    gid = k * _CK + lax.broadcasted_iota(jnp.int32, (1, _CK), 1)
    prod = t_ref[...] * csum
    prod = jnp.where(gid < vocab, prod, 0.0)
    psum = jnp.sum(prod, axis=1, keepdims=True)

    @pl.when(k == 0)
    def _():
        acc_ref[...] = jnp.zeros_like(acc_ref)

    acc_ref[...] += psum

    @pl.when(k == nblk - 1)
    def _():
        bow = acc_ref[...]
        logits = (
            jnp.dot(w_ref[...], bow, preferred_element_type=jnp.float32)
            + b_ref[...]
        )
        m = jnp.max(logits, axis=0, keepdims=True)
        e = jnp.exp(logits - m)
        lse = jnp.log(jnp.sum(e, axis=0, keepdims=True)) + m
        o_ref[...] = logits - lse


def kernel(inputs, emb_table, W, b):
    n_tokens = inputs.shape[0]
    vocab = emb_table.shape[0]
    num_labels = W.shape[0]

    per_w = n_tokens // _NW
    idx = inputs.astype(jnp.int32).reshape(_NW, per_w // _SCHUNK, _SCHUNK)
    zeros = jnp.zeros((vocab,), jnp.float32)

    cnt2 = _make_count(n_tokens, vocab)(idx, zeros)

    table_t = emb_table.T

    nblk = (vocab + _CK - 1) // _CK
    out_col = pl.pallas_call(
        functools.partial(_matvec_head_kernel, nblk, vocab),
        grid=(nblk,),
        in_specs=[
            pl.BlockSpec((_NC, _CK), lambda k: (0, k)),
            pl.BlockSpec((_D, _CK), lambda k: (0, k)),
            pl.BlockSpec((num_labels, _D), lambda k: (0, 0)),
            pl.BlockSpec((num_labels, 1), lambda k: (0, 0)),
        ],
        out_specs=pl.BlockSpec((num_labels, 1), lambda k: (0, 0)),
        out_shape=jax.ShapeDtypeStruct((num_labels, 1), jnp.float32),
        scratch_shapes=[pltpu.VMEM((_D, 1), jnp.float32)],
    )(cnt2, table_t, W, b.reshape(num_labels, 1))

    return out_col.reshape(1, num_labels)


# speedup vs baseline: 4.8520x; 1.0137x over previous
import functools

import jax
import jax.numpy as jnp
from jax import lax
from jax.experimental import pallas as pl
from jax.experimental.pallas import tpu as pltpu
from jax.experimental.pallas import tpu_sc as plsc

_D = 64
_LANES = 16

_NC = 2
_NS = 16
_NW = _NC * _NS

_SCHUNK = 128

_STRIPE = 62464

_CK = 32768


def _make_count(n_tokens: int, vocab: int):
    per_w = n_tokens // _NW
    nch = per_w // _SCHUNK
    last_stripe = vocab - (_NS - 1) * _STRIPE

    mesh = plsc.VectorSubcoreMesh(core_axis_name="c", subcore_axis_name="s")

    @functools.partial(
        pl.kernel,
        out_type=jax.ShapeDtypeStruct((_NC, vocab), jnp.float32),
        mesh=mesh,
        scratch_types=[
            pltpu.VMEM((nch, _SCHUNK), jnp.int32),
            pltpu.VMEM((_SCHUNK,), jnp.float32),
            pltpu.VMEM((last_stripe,), jnp.float32),
            pltpu.VMEM_SHARED((vocab,), jnp.float32),
            pltpu.SemaphoreType.DMA,
            pltpu.SemaphoreType.DMA,
        ],
    )
    def count(idx_hbm, out_hbm, idx_v, ones_v, zero_v, cnt_sh, sem_i, sem_z):
        cid = lax.axis_index("c")
        sid = lax.axis_index("s")
        wid = sid * _NC + cid

        ci = pltpu.async_copy(idx_hbm.at[wid], idx_v, sem_i)

        for g in range(_SCHUNK // _LANES):
            ones_v[pl.ds(g * _LANES, _LANES)] = jnp.ones(
                (_LANES,), jnp.float32
            )

        zeros16 = jnp.zeros((_LANES,), jnp.float32)

        def zbody(i, _):
            zero_v[pl.ds(i * _LANES, _LANES)] = zeros16
            return 0

        lax.fori_loop(0, last_stripe // _LANES, zbody, 0)

        base = sid * _STRIPE

        @pl.when(sid == _NS - 1)
        def _():
            pltpu.async_copy(
                zero_v, cnt_sh.at[pl.ds(base, last_stripe)], sem_z
            ).wait()

        @pl.when(sid != _NS - 1)
        def _():
            pltpu.async_copy(
                zero_v.at[pl.ds(0, _STRIPE)],
                cnt_sh.at[pl.ds(base, _STRIPE)],
                sem_z,
            ).wait()

        ci.wait()
        plsc.subcore_barrier()

        for j in range(nch):
            pltpu.sync_copy(ones_v, cnt_sh.at[idx_v.at[j]], add=True)

        plsc.subcore_barrier()

        @pl.when(sid == 0)
        def _():
            pltpu.sync_copy(cnt_sh, out_hbm.at[cid])

    return count


def _matvec_head_kernel(nblk, vocab, cnt_ref, t_ref, w_ref, b_ref, o_ref,
                        acc_ref):
    k = pl.program_id(0)
    csum = cnt_ref[0:1, :] + cnt_ref[1:2, :]
    t = t_ref[...]

    def tail_psum(_):
        gid = (nblk - 1) * _CK + lax.broadcasted_iota(
            jnp.int32, (1, _CK), 1
        )
        prod = jnp.where(gid < vocab, t * csum, 0.0)
        return jnp.sum(prod, axis=1, keepdims=True)

    def main_psum(_):
        return jnp.sum(t * csum, axis=1, keepdims=True)

    psum = lax.cond(k == nblk - 1, tail_psum, main_psum, 0)

    @pl.when(k == 0)
    def _():
        acc_ref[...] = jnp.zeros_like(acc_ref)

    acc_ref[...] += psum

    @pl.when(k == nblk - 1)
    def _():
        bow = acc_ref[...]
        logits = (
            jnp.dot(w_ref[...], bow, preferred_element_type=jnp.float32)
            + b_ref[...]
        )
        m = jnp.max(logits, axis=0, keepdims=True)
        e = jnp.exp(logits - m)
        lse = jnp.log(jnp.sum(e, axis=0, keepdims=True)) + m
        o_ref[...] = logits - lse


def kernel(inputs, emb_table, W, b):
    n_tokens = inputs.shape[0]
    vocab = emb_table.shape[0]
    num_labels = W.shape[0]

    per_w = n_tokens // _NW
    idx = inputs.astype(jnp.int32).reshape(_NW, per_w // _SCHUNK, _SCHUNK)

    cnt2 = _make_count(n_tokens, vocab)(idx)

    table_t = emb_table.T

    nblk = (vocab + _CK - 1) // _CK
    out_col = pl.pallas_call(
        functools.partial(_matvec_head_kernel, nblk, vocab),
        grid=(nblk,),
        in_specs=[
            pl.BlockSpec((_NC, _CK), lambda k: (0, k)),
            pl.BlockSpec((_D, _CK), lambda k: (0, k)),
            pl.BlockSpec((num_labels, _D), lambda k: (0, 0)),
            pl.BlockSpec((num_labels, 1), lambda k: (0, 0)),
        ],
        out_specs=pl.BlockSpec((num_labels, 1), lambda k: (0, 0)),
        out_shape=jax.ShapeDtypeStruct((num_labels, 1), jnp.float32),
        scratch_shapes=[pltpu.VMEM((_D, 1), jnp.float32)],
    )(cnt2, table_t, W, b.reshape(num_labels, 1))

    return out_col.reshape(1, num_labels)
